# Initial kernel scaffold; baseline (speedup 1.0000x reference)
import jax
import jax.numpy as jnp
from jax.experimental import pallas as pl


def kernel(activations, sample_points, agg_weights):
    raise NotImplementedError("write your pallas kernel here")


# speedup vs baseline: 9.8826x; 9.8826x over previous
import functools

import jax
import jax.numpy as jnp
from jax import lax
from jax.experimental import pallas as pl
from jax.experimental.pallas import tpu as pltpu
from jax.experimental.pallas import tpu_sc as plsc

B = 64
N_IN = 8192
N_OUT = 8192
DEG = 16

NC = 2
NS = 16
NW = NC * NS
L = 16

O_PER = N_OUT // NW
G = 8
NG = O_PER // G
ROWS = G * DEG
SLAB = 2 * B


def _sc_body(t2_hbm, sp_hbm, w_hbm, out_hbm,
             sp_v, w_v, idx_v, wlo_v, whi_v, gbuf, obuf, sem):
    wid = lax.axis_index("s") * NC + lax.axis_index("c")
    obase = wid * O_PER

    pltpu.sync_copy(sp_hbm.at[pl.ds(obase, O_PER)], sp_v)
    pltpu.sync_copy(w_hbm.at[pl.ds(obase, O_PER)], w_v)

    def prep(o, _):
        sp = sp_v[o]
        w = w_v[o]
        c = jnp.clip(sp, 0.0, 1.0) * float(N_IN - 1)
        lo = c.astype(jnp.int32)
        f = c - lo.astype(jnp.float32)
        idx_v[pl.ds(o * DEG, DEG)] = lo
        wlo_v[pl.ds(o * DEG, DEG)] = w * (1.0 - f)
        whi_v[pl.ds(o * DEG, DEG)] = w * f
        return 0

    lax.fori_loop(0, O_PER, prep, 0)

    def group(g, _):
        cp = pltpu.async_copy(t2_hbm.at[idx_v.at[pl.ds(g * ROWS, ROWS)]],
                              gbuf, sem)
        cp.wait()

        def one_out(om, _):
            o = g * G + om
            wlo_vec = wlo_v[pl.ds(o * DEG, DEG)]
            whi_vec = whi_v[pl.ds(o * DEG, DEG)]
            acc = [jnp.zeros((L,), jnp.float32) for _ in range(B // L)]
            for d in range(DEG):
                r = om * DEG + d
                wlo = wlo_vec[d]
                whi = whi_vec[d]
                for k in range(B // L):
                    acc[k] = acc[k] + wlo * gbuf[r, pl.ds(k * L, L)]
                    acc[k] = acc[k] + whi * gbuf[r, pl.ds(B + k * L, L)]
            for k in range(B // L):
                obuf[o, pl.ds(k * L, L)] = acc[k]
            return 0

        lax.fori_loop(0, G, one_out, 0)
        return 0

    lax.fori_loop(0, NG, group, 0)

    pltpu.sync_copy(obuf, out_hbm.at[pl.ds(obase, O_PER)])


@jax.jit
def _run(t2, sp, w):
    mesh = plsc.VectorSubcoreMesh(core_axis_name="c", subcore_axis_name="s")
    return pl.kernel(
        _sc_body,
        out_type=jax.ShapeDtypeStruct((N_OUT, B), jnp.float32),
        mesh=mesh,
        scratch_types=[
            pltpu.VMEM((O_PER, DEG), jnp.float32),
            pltpu.VMEM((O_PER, DEG), jnp.float32),
            pltpu.VMEM((O_PER * DEG,), jnp.int32),
            pltpu.VMEM((O_PER * DEG,), jnp.float32),
            pltpu.VMEM((O_PER * DEG,), jnp.float32),
            pltpu.VMEM((ROWS, SLAB), jnp.float32),
            pltpu.VMEM((O_PER, B), jnp.float32),
            pltpu.SemaphoreType.DMA,
        ],
    )(t2, sp, w)


def kernel(activations, sample_points, agg_weights):
    a_t = activations.T
    nxt = jnp.concatenate([a_t[1:], a_t[-1:]], axis=0)
    t2 = jnp.concatenate([a_t, nxt], axis=1)
    sp = sample_points[..., 0]
    out_t = _run(t2, sp, agg_weights)
    return out_t.T


# speedup vs baseline: 12.7354x; 1.2887x over previous
import functools

import jax
import jax.numpy as jnp
from jax import lax
from jax.experimental import pallas as pl
from jax.experimental.pallas import tpu as pltpu
from jax.experimental.pallas import tpu_sc as plsc

B = 64
N_IN = 8192
N_OUT = 8192
DEG = 16

NC = 2
NS = 16
NW = NC * NS
L = 16

O_PER = N_OUT // NW
G = 8
NG = O_PER // G
ROWS = G * DEG
SLAB = 2 * B


def _sc_body(t2_hbm, sp_hbm, w_hbm, out_hbm,
             sp_v, w_v, idx_v, wlo_v, whi_v, gbuf, gbuf2, obuf, sem, sem2):
    wid = lax.axis_index("s") * NC + lax.axis_index("c")
    obase = wid * O_PER

    pltpu.sync_copy(sp_hbm.at[pl.ds(obase * DEG, O_PER * DEG)], sp_v)
    pltpu.sync_copy(w_hbm.at[pl.ds(obase * DEG, O_PER * DEG)], w_v)

    def prep(o, _):
        sp = sp_v[pl.ds(o * DEG, DEG)]
        w = w_v[pl.ds(o * DEG, DEG)]
        c = jnp.clip(sp, 0.0, 1.0) * float(N_IN - 1)
        lo = c.astype(jnp.int32)
        f = c - lo.astype(jnp.float32)
        idx_v[pl.ds(o * DEG, DEG)] = lo
        wlo_v[pl.ds(o * DEG, DEG)] = w * (1.0 - f)
        whi_v[pl.ds(o * DEG, DEG)] = w * f
        return 0

    lax.fori_loop(0, O_PER, prep, 0)

    def compute_group(g, buf):
        def one_out(om, _):
            o = g * G + om
            wlo_vec = wlo_v[pl.ds(o * DEG, DEG)]
            whi_vec = whi_v[pl.ds(o * DEG, DEG)]
            acc = [jnp.zeros((L,), jnp.float32) for _ in range(B // L)]
            for d in range(DEG):
                r = om * DEG + d
                wlo = wlo_vec[d]
                whi = whi_vec[d]
                for k in range(B // L):
                    acc[k] = acc[k] + wlo * buf[r, pl.ds(k * L, L)]
                    acc[k] = acc[k] + whi * buf[r, pl.ds(B + k * L, L)]
            for k in range(B // L):
                obuf[pl.ds(o * B + k * L, L)] = acc[k]
            return 0

        lax.fori_loop(0, G, one_out, 0)

    def start_gather(g, buf, sem):
        return pltpu.async_copy(
            t2_hbm.at[idx_v.at[pl.ds(g * ROWS, ROWS)]], buf, sem)

    def do_group(g, buf, sem, nbuf, nsem):
        @pl.when(g + 1 < NG)
        def _():
            start_gather(g + 1, nbuf, nsem)
        pltpu.make_async_copy(
            t2_hbm.at[idx_v.at[pl.ds(g * ROWS, ROWS)]], buf, sem).wait()
        compute_group(g, buf)

    start_gather(0, gbuf, sem)

    def group(g, _):
        @pl.when(g % 2 == 0)
        def _():
            do_group(g, gbuf, sem, gbuf2, sem2)

        @pl.when(g % 2 == 1)
        def _():
            do_group(g, gbuf2, sem2, gbuf, sem)
        return 0

    lax.fori_loop(0, NG, group, 0)

    pltpu.sync_copy(obuf, out_hbm.at[pl.ds(obase * B, O_PER * B)])


@jax.jit
def _run(t2, sp, w):
    mesh = plsc.VectorSubcoreMesh(core_axis_name="c", subcore_axis_name="s")
    return pl.kernel(
        _sc_body,
        out_type=jax.ShapeDtypeStruct((N_OUT * B,), jnp.float32),
        mesh=mesh,
        scratch_types=[
            pltpu.VMEM((O_PER * DEG,), jnp.float32),
            pltpu.VMEM((O_PER * DEG,), jnp.float32),
            pltpu.VMEM((O_PER * DEG,), jnp.int32),
            pltpu.VMEM((O_PER * DEG,), jnp.float32),
            pltpu.VMEM((O_PER * DEG,), jnp.float32),
            pltpu.VMEM((ROWS, SLAB), jnp.float32),
            pltpu.VMEM((ROWS, SLAB), jnp.float32),
            pltpu.VMEM((O_PER * B,), jnp.float32),
            pltpu.SemaphoreType.DMA,
            pltpu.SemaphoreType.DMA,
        ],
    )(t2, sp, w)


def kernel(activations, sample_points, agg_weights):
    a_t = activations.T
    nxt = jnp.concatenate([a_t[1:], a_t[-1:]], axis=0)
    t2 = jnp.concatenate([a_t, nxt], axis=1)
    sp = sample_points.reshape(-1)
    out_t = _run(t2, sp, agg_weights.reshape(-1))
    return out_t.reshape(N_OUT, B).T


# speedup vs baseline: 13.1287x; 1.0309x over previous
import functools

import jax
import jax.numpy as jnp
from jax import lax
from jax.experimental import pallas as pl
from jax.experimental.pallas import tpu as pltpu
from jax.experimental.pallas import tpu_sc as plsc

B = 64
N_IN = 8192
N_OUT = 8192
DEG = 16

NC = 2
NS = 16
NW = NC * NS
L = 16

O_PER = N_OUT // NW
G = 8
NG = O_PER // G
ROWS = G * DEG
SLAB = 2 * B


def _sc_body(t2_hbm, sp_hbm, w_hbm, out_hbm,
             sp_v, w_v, idx_v, wlo_v, whi_v, gbuf, gbuf2, obuf, obuf2,
             sem, sem2, osem):
    wid = lax.axis_index("s") * NC + lax.axis_index("c")
    obase = wid * O_PER

    pltpu.sync_copy(sp_hbm.at[pl.ds(obase * DEG, O_PER * DEG)], sp_v)
    pltpu.sync_copy(w_hbm.at[pl.ds(obase * DEG, O_PER * DEG)], w_v)

    def prep(o, _):
        sp = sp_v[pl.ds(o * DEG, DEG)]
        w = w_v[pl.ds(o * DEG, DEG)]
        c = jnp.clip(sp, 0.0, 1.0) * float(N_IN - 1)
        lo = c.astype(jnp.int32)
        f = c - lo.astype(jnp.float32)
        idx_v[pl.ds(o * DEG, DEG)] = lo
        wlo_v[pl.ds(o * DEG, DEG)] = w * (1.0 - f)
        whi_v[pl.ds(o * DEG, DEG)] = w * f
        return 0

    lax.fori_loop(0, O_PER, prep, 0)

    def compute_group(g, buf):
        def one_out(om, _):
            o = g * G + om
            wlo_vec = wlo_v[pl.ds(o * DEG, DEG)]
            whi_vec = whi_v[pl.ds(o * DEG, DEG)]
            acc = [jnp.zeros((L,), jnp.float32) for _ in range(B // L)]
            for d in range(DEG):
                r = om * DEG + d
                wlo = wlo_vec[d]
                whi = whi_vec[d]
                for k in range(B // L):
                    acc[k] = acc[k] + wlo * buf[r, pl.ds(k * L, L)]
                    acc[k] = acc[k] + whi * buf[r, pl.ds(B + k * L, L)]
            for k in range(B // L):
                obuf[pl.ds(o * B + k * L, L)] = acc[k]
            return 0

        lax.fori_loop(0, G, one_out, 0)

    def start_gather(g, buf, sem):
        return pltpu.async_copy(
            t2_hbm.at[idx_v.at[pl.ds(g * ROWS, ROWS)]], buf, sem)

    def do_group(g, buf, sem, nbuf, nsem):
        @pl.when(g + 1 < NG)
        def _():
            start_gather(g + 1, nbuf, nsem)
        pltpu.make_async_copy(
            t2_hbm.at[idx_v.at[pl.ds(g * ROWS, ROWS)]], buf, sem).wait()
        compute_group(g, buf)

    start_gather(0, gbuf, sem)

    def group(g, _):
        @pl.when(g % 2 == 0)
        def _():
            do_group(g, gbuf, sem, gbuf2, sem2)

        @pl.when(g % 2 == 1)
        def _():
            do_group(g, gbuf2, sem2, gbuf, sem)
        return 0

    lax.fori_loop(0, NG, group, 0)

    stride_idx = lax.iota(jnp.int32, L) * B

    def trans_b(b, _):
        def trans_j(j, _):
            v = plsc.load_gather(obuf, [stride_idx + (j * L * B + b)])
            obuf2[pl.ds(b * O_PER + j * L, L)] = v
            return 0
        lax.fori_loop(0, O_PER // L, trans_j, 0)
        return 0

    lax.fori_loop(0, B, trans_b, 0)

    for b in range(B):
        pltpu.async_copy(obuf2.at[pl.ds(b * O_PER, O_PER)],
                         out_hbm.at[b, pl.ds(obase, O_PER)], osem)
    for b in range(B):
        pltpu.make_async_copy(obuf2.at[pl.ds(b * O_PER, O_PER)],
                              out_hbm.at[b, pl.ds(obase, O_PER)], osem).wait()


@jax.jit
def _run(t2, sp, w):
    mesh = plsc.VectorSubcoreMesh(core_axis_name="c", subcore_axis_name="s")
    return pl.kernel(
        _sc_body,
        out_type=jax.ShapeDtypeStruct((B, N_OUT), jnp.float32),
        mesh=mesh,
        compiler_params=pltpu.CompilerParams(needs_layout_passes=False),
        scratch_types=[
            pltpu.VMEM((O_PER * DEG,), jnp.float32),
            pltpu.VMEM((O_PER * DEG,), jnp.float32),
            pltpu.VMEM((O_PER * DEG,), jnp.int32),
            pltpu.VMEM((O_PER * DEG,), jnp.float32),
            pltpu.VMEM((O_PER * DEG,), jnp.float32),
            pltpu.VMEM((ROWS, SLAB), jnp.float32),
            pltpu.VMEM((ROWS, SLAB), jnp.float32),
            pltpu.VMEM((B * O_PER,), jnp.float32),
            pltpu.VMEM((B * O_PER,), jnp.float32),
            pltpu.SemaphoreType.DMA,
            pltpu.SemaphoreType.DMA,
            pltpu.SemaphoreType.DMA,
        ],
    )(t2, sp, w)


def _t2_body(a_ref, t2_ref):
    at = a_ref[...].T
    nxt = jnp.concatenate([at[1:], at[-1:]], axis=0)
    t2_ref[...] = jnp.concatenate([at, nxt], axis=1)


@jax.jit
def _build_t2(activations):
    return pl.pallas_call(
        _t2_body,
        out_shape=jax.ShapeDtypeStruct((N_IN, 2 * B), jnp.float32),
    )(activations)


def kernel(activations, sample_points, agg_weights):
    t2 = _build_t2(activations)
    sp = sample_points.reshape(-1)
    return _run(t2, sp, agg_weights.reshape(-1))
